# baseline (speedup 1.0000x reference)
import functools

import jax
import jax.numpy as jnp
from jax import lax
from jax.experimental import pallas as pl
from jax.experimental.pallas import tpu as pltpu
from jax.experimental.pallas import tpu_sc as plsc


def _build_gather(n_rows: int, d: int, n_table_rows: int,
                  chunk_b: int, frac_a_16ths: int):
    info = plsc.get_sparse_core_info()
    nc, ns = info.num_cores, info.num_subcores
    nw = nc * ns
    assert n_rows % nw == 0
    per_w = n_rows // nw

    n_a = (per_w * frac_a_16ths // 16) // 32 * 32
    n_b = per_w - n_a
    assert n_b % chunk_b == 0
    n_chunks = n_b // chunk_b
    assert n_chunks % 2 == 0 and n_chunks >= 4
    ka_groups = n_a // 16 // n_chunks
    ka_rem_groups = n_a // 16 - ka_groups * n_chunks

    mesh = plsc.VectorSubcoreMesh(core_axis_name="c", subcore_axis_name="s")

    @functools.partial(
        pl.kernel,
        mesh=mesh,
        out_type=jax.ShapeDtypeStruct((n_rows, d), jnp.float32),
        scratch_types=[
            pltpu.VMEM((per_w,), jnp.int32),
            pltpu.VMEM((chunk_b, d), jnp.float32),
            pltpu.VMEM((chunk_b, d), jnp.float32),
            pltpu.VMEM_SHARED((n_table_rows, d), jnp.float32),
            pltpu.SemaphoreType.DMA,
            pltpu.SemaphoreType.DMA,
            pltpu.SemaphoreType.DMA,
            pltpu.SemaphoreType.DMA,
            pltpu.SemaphoreType.DMA,
        ],
    )
    def gather_kernel(idx_hbm, table_hbm, out_hbm, idx_v, rows_a, rows_b,
                      table_sp, gsem_a, gsem_b, osem_a, osem_b, asem):
        wid = lax.axis_index("s") * nc + lax.axis_index("c")
        base = wid * per_w

        @pl.when(lax.axis_index("s") == 0)
        def _():
            pltpu.sync_copy(table_hbm, table_sp)

        pltpu.sync_copy(idx_hbm.at[pl.ds(base, per_w)], idx_v)
        plsc.subcore_barrier()

        bufs = ((rows_a, gsem_a, osem_a), (rows_b, gsem_b, osem_b))

        def b_idx(c):
            return idx_v.at[pl.ds(n_a + c * chunk_b, chunk_b)]

        def start_gather(c, rows, gsem):
            pltpu.async_copy(table_hbm.at[b_idx(c)], rows, gsem)

        def wait_gather(c, rows, gsem):
            pltpu.make_async_copy(table_hbm.at[b_idx(c)], rows, gsem).wait()

        def out_slice(c):
            return out_hbm.at[pl.ds(base + n_a + c * chunk_b, chunk_b)]

        def start_out(c, rows, osem):
            pltpu.async_copy(rows, out_slice(c), osem)

        def wait_out(c, rows, osem):
            pltpu.make_async_copy(rows, out_slice(c), osem).wait()

        def fire_a_group(g):
            vec = idx_v[pl.ds(g * 16, 16)]
            for l in range(16):
                pltpu.async_copy(table_sp.at[vec[l]],
                                 out_hbm.at[base + g * 16 + l], asem)

        start_gather(0, rows_a, gsem_a)
        start_gather(1, rows_b, gsem_b)

        def body(p, carry):
            for b, (rows, gsem, osem) in enumerate(bufs):
                c = 2 * p + b
                wait_gather(c, rows, gsem)
                start_out(c, rows, osem)
                for g in range(ka_groups):
                    fire_a_group(c * ka_groups + g)
                wait_out(c, rows, osem)
                start_gather(c + 2, rows, gsem)
            return carry

        lax.fori_loop(0, n_chunks // 2 - 1, body, 0)

        for b, (rows, gsem, osem) in enumerate(bufs):
            c = n_chunks - 2 + b
            wait_gather(c, rows, gsem)
            start_out(c, rows, osem)
            for g in range(ka_groups):
                fire_a_group(c * ka_groups + g)
            wait_out(c, rows, osem)

        def rem(g, carry):
            fire_a_group(n_chunks * ka_groups + g)
            return carry

        lax.fori_loop(0, ka_rem_groups, rem, 0)

        def drain(c, carry):
            pltpu.make_async_copy(table_sp.at[0], out_hbm.at[base], asem).wait()
            return carry

        lax.fori_loop(0, n_a, drain, 0)

    return gather_kernel


def kernel(indices, bank_embedding_weight):
    b, s = indices.shape
    v, d = bank_embedding_weight.shape
    n = b * s
    flat = indices.reshape(n).astype(jnp.int32)
    out = _build_gather(n, d, n_table_rows=v, chunk_b=16,
                        frac_a_16ths=8)(flat, bank_embedding_weight)
    return out.reshape(b, s, d)


# speedup vs baseline: 1.0680x; 1.0680x over previous
import functools

import jax
import jax.numpy as jnp
from jax import lax
from jax.experimental import pallas as pl
from jax.experimental.pallas import tpu as pltpu
from jax.experimental.pallas import tpu_sc as plsc


def _build_gather(n_rows: int, d: int, n_table_rows: int):
    chunk = 16
    info = plsc.get_sparse_core_info()
    nc, ns = info.num_cores, info.num_subcores
    nw = nc * ns
    assert n_rows % nw == 0
    per_w = n_rows // nw
    assert per_w % chunk == 0
    n_chunks = per_w // chunk
    assert n_chunks % 2 == 0 and n_chunks >= 4

    mesh = plsc.VectorSubcoreMesh(core_axis_name="c", subcore_axis_name="s")

    @functools.partial(
        pl.kernel,
        mesh=mesh,
        out_type=jax.ShapeDtypeStruct((n_rows, d), jnp.float32),
        scratch_types=[
            pltpu.VMEM((per_w,), jnp.int32),
            pltpu.VMEM((chunk, d), jnp.float32),
            pltpu.VMEM((chunk, d), jnp.float32),
            pltpu.VMEM_SHARED((n_table_rows, d), jnp.float32),
            pltpu.SemaphoreType.DMA,
            pltpu.SemaphoreType.DMA,
            pltpu.SemaphoreType.DMA,
            pltpu.SemaphoreType.DMA,
        ],
    )
    def gather_kernel(idx_hbm, table_hbm, out_hbm, idx_v, rows_a, rows_b,
                      table_sp, fsem_a, fsem_b, osem_a, osem_b):
        wid = lax.axis_index("s") * nc + lax.axis_index("c")
        base = wid * per_w

        @pl.when(lax.axis_index("s") == 0)
        def _():
            pltpu.sync_copy(table_hbm, table_sp)

        pltpu.sync_copy(idx_hbm.at[pl.ds(base, per_w)], idx_v)
        plsc.subcore_barrier()

        bufs = ((rows_a, fsem_a, osem_a), (rows_b, fsem_b, osem_b))

        def fill(c, rows, fsem):
            vec = idx_v[pl.ds(c * chunk, chunk)]
            for l in range(chunk):
                pltpu.async_copy(table_sp.at[vec[l]], rows.at[l], fsem)
            pltpu.make_async_copy(table_hbm.at[pl.ds(0, chunk)], rows,
                                  fsem).wait()

        def out_slice(c):
            return out_hbm.at[pl.ds(base + c * chunk, chunk)]

        def start_out(c, rows, osem):
            pltpu.async_copy(rows, out_slice(c), osem)

        def wait_out(c, rows, osem):
            pltpu.make_async_copy(rows, out_slice(c), osem).wait()

        for b, (rows, fsem, osem) in enumerate(bufs):
            fill(b, rows, fsem)
            start_out(b, rows, osem)

        def body(p, carry):
            for b, (rows, fsem, osem) in enumerate(bufs):
                c = 2 * p + b
                wait_out(c - 2, rows, osem)
                fill(c, rows, fsem)
                start_out(c, rows, osem)
            return carry

        lax.fori_loop(1, n_chunks // 2, body, 0)

        for b, (rows, fsem, osem) in enumerate(bufs):
            c = n_chunks - 2 + b
            wait_out(c, rows, osem)

    return gather_kernel


def kernel(indices, bank_embedding_weight):
    b, s = indices.shape
    v, d = bank_embedding_weight.shape
    n = b * s
    flat = indices.reshape(n).astype(jnp.int32)
    out = _build_gather(n, d, n_table_rows=v)(flat, bank_embedding_weight)
    return out.reshape(b, s, d)
